# Initial kernel scaffold; baseline (speedup 1.0000x reference)
import jax
import jax.numpy as jnp
from jax.experimental import pallas as pl


def kernel(x, edge_index, W_init, W1, b1, W2, b2):
    raise NotImplementedError("write your pallas kernel here")


# speedup vs baseline: 4.4167x; 4.4167x over previous
import functools

import jax
import jax.numpy as jnp
from jax import lax
from jax.experimental import pallas as pl
from jax.experimental.pallas import tpu as pltpu
from jax.experimental.pallas import tpu_sc as plsc

_N = 10000
_D = 128
_E = 320000
_EPS = 1e-5

_NC = 2
_NS = 16
_CHUNK = 128
_CH = -(-_E // (_NC * _NS * _CHUNK))
_EPT = _CH * _CHUNK
_EPAD = _NC * _NS * _EPT
_NPAD = 10240
_ZRPT = _NPAD // _NS
_ZCH = _ZRPT // _CHUNK
_ORPT = _N // _NS


def _fc_bn_relu_body(x_ref, w_ref, b_ref, o_ref):
    y = jnp.dot(x_ref[...], w_ref[...], preferred_element_type=jnp.float32)
    y = y + b_ref[...]
    mean = jnp.mean(y, axis=0, keepdims=True)
    cen = y - mean
    var = jnp.mean(cen * cen, axis=0, keepdims=True)
    o_ref[...] = jnp.maximum(cen * lax.rsqrt(var + _EPS), 0.0)


def _sum_fc_bn_relu_body(p_ref, w_ref, b_ref, o_ref):
    a = p_ref[0, pl.ds(0, _N), :] + p_ref[1, pl.ds(0, _N), :]
    y = jnp.dot(a, w_ref[...], preferred_element_type=jnp.float32)
    y = y + b_ref[...]
    mean = jnp.mean(y, axis=0, keepdims=True)
    cen = y - mean
    var = jnp.mean(cen * cen, axis=0, keepdims=True)
    o_ref[...] = jnp.maximum(cen * lax.rsqrt(var + _EPS), 0.0)


_dense_in = pl.pallas_call(
    _fc_bn_relu_body,
    out_shape=jax.ShapeDtypeStruct((_N, _D), jnp.float32),
)

_dense_agg = pl.pallas_call(
    _sum_fc_bn_relu_body,
    out_shape=jax.ShapeDtypeStruct((_N, _D), jnp.float32),
)


@functools.partial(
    pl.kernel,
    mesh=plsc.VectorSubcoreMesh(core_axis_name="c", subcore_axis_name="s"),
    out_type=jax.ShapeDtypeStruct((_NC, _NPAD, _D), jnp.float32),
    scratch_types=[
        pltpu.VMEM((_CH, _CHUNK), jnp.int32),
        pltpu.VMEM((_CH, _CHUNK), jnp.int32),
        pltpu.VMEM((_CHUNK, _D), jnp.float32),
        pltpu.VMEM_SHARED((_NPAD, _D), jnp.float32),
        pltpu.SemaphoreType.DMA,
    ],
)
def _sc_agg(h_hbm, src_hbm, dst_hbm, z_hbm, out_hbm,
            src_v, dst_v, rows_v, agg_sh, sem):
    c = lax.axis_index("c")
    s = lax.axis_index("s")
    pltpu.sync_copy(z_hbm, rows_v)
    for k in range(_ZCH):
        pltpu.sync_copy(rows_v, agg_sh.at[pl.ds(s * _ZRPT + k * _CHUNK, _CHUNK)])
    plsc.subcore_barrier()
    pltpu.sync_copy(src_hbm.at[c, s], src_v)
    pltpu.sync_copy(dst_hbm.at[c, s], dst_v)

    def body(j, carry):
        pltpu.async_copy(h_hbm.at[src_v.at[j]], rows_v, sem).wait()
        pltpu.sync_copy(rows_v, agg_sh.at[dst_v.at[j]], add=True)
        return carry

    lax.fori_loop(0, _CH, body, 0)
    plsc.subcore_barrier()
    pltpu.sync_copy(agg_sh.at[pl.ds(s * _ZRPT, _ZRPT)],
                    out_hbm.at[c, pl.ds(s * _ZRPT, _ZRPT)])


def kernel(x, edge_index, W_init, W1, b1, W2, b2):
    src = edge_index[0]
    dst = edge_index[1]
    pad = _EPAD - _E
    src_p = jnp.concatenate(
        [src, jnp.zeros((pad,), jnp.int32)]).reshape(_NC, _NS, _CH, _CHUNK)
    dst_p = jnp.concatenate(
        [dst, jnp.full((pad,), _N, jnp.int32)]).reshape(_NC, _NS, _CH, _CHUNK)
    z = jnp.zeros((_CHUNK, _D), jnp.float32)
    b0 = jnp.zeros((1, _D), jnp.float32)

    h = _dense_in(x, W_init, b0)
    p = _sc_agg(h, src_p, dst_p, z)
    h = _dense_agg(p, W1, b1.reshape(1, _D))
    p = _sc_agg(h, src_p, dst_p, z)
    h = _dense_agg(p, W2, b2.reshape(1, _D))
    return h
